# baseline (speedup 1.0000x reference)
import jax
import jax.numpy as jnp
from jax.experimental import pallas as pl
from jax.experimental.pallas import tpu as pltpu

_H = 320
_W = 320
_CONTRAST = 4
_COIL = 12
_COIL_CHUNK = 4
_SLOPE = 5.0
_CENTER = 10
_R = 4.0


def _fused_body(w_ref, noise_ref, im_ref, lam_ref, inv_ref, mask_ref):
    j = pl.program_id(1)

    @pl.when(j == 0)
    def _compute_mask():
        w = w_ref[0]
        prob = jax.nn.sigmoid(w * _SLOPE)
        rows = jax.lax.broadcasted_iota(jnp.int32, (_H, _W), 0)
        cols = jax.lax.broadcasted_iota(jnp.int32, (_H, _W), 1)
        cy0, cy1 = _H // 2 - _CENTER // 2, _H // 2 + _CENTER // 2
        cx0, cx1 = _W // 2 - _CENTER // 2, _W // 2 + _CENTER // 2
        in_center = (rows >= cy0) & (rows < cy1) & (cols >= cx0) & (cols < cx1)
        p = jnp.where(in_center, 0.0, prob)
        s = jnp.sum(p)
        total = _H * _W / _R - _CENTER ** 2
        p_over = p * (total / s)
        inv_total = _H * _W * (1.0 - 1.0 / _R)
        inv_sum = _H * _W - s - _CENTER ** 2
        p_under = 1.0 - (1.0 - p) * (inv_total / inv_sum)
        p_new = jnp.where(s > total, p_over, p_under)
        p_new = jnp.where(in_center, 1.0, p_new)
        mask_ref[...] = (p_new - noise_ref[0] >= 0.0).astype(jnp.float32)

    m = mask_ref[...]
    im = im_ref[0]
    lam = im * m[None, :, :]
    lam_ref[0] = lam
    inv_ref[0] = im - lam


def kernel(undersampled_k, initial_mask, sampling_weights):
    batch = undersampled_k.shape[0]
    noise = jax.random.uniform(
        jax.random.key(42), (batch, _CONTRAST, _H, _W), dtype=jnp.float32
    )
    bc = batch * _CONTRAST
    im = initial_mask.reshape(bc, _COIL, _H, _W)
    noise_f = noise.reshape(bc, _H, _W)
    ncs = _COIL // _COIL_CHUNK

    lam, inv = pl.pallas_call(
        _fused_body,
        grid=(bc, ncs),
        in_specs=[
            pl.BlockSpec((1, _H, _W), lambda i, j: (i % _CONTRAST, 0, 0)),
            pl.BlockSpec((1, _H, _W), lambda i, j: (i, 0, 0)),
            pl.BlockSpec((1, _COIL_CHUNK, _H, _W), lambda i, j: (i, j, 0, 0)),
        ],
        out_specs=[
            pl.BlockSpec((1, _COIL_CHUNK, _H, _W), lambda i, j: (i, j, 0, 0)),
            pl.BlockSpec((1, _COIL_CHUNK, _H, _W), lambda i, j: (i, j, 0, 0)),
        ],
        out_shape=[
            jax.ShapeDtypeStruct((bc, _COIL, _H, _W), jnp.float32),
            jax.ShapeDtypeStruct((bc, _COIL, _H, _W), jnp.float32),
        ],
        scratch_shapes=[pltpu.VMEM((_H, _W), jnp.float32)],
        compiler_params=pltpu.CompilerParams(
            dimension_semantics=("parallel", "arbitrary"),
        ),
    )(sampling_weights, noise_f, im)

    shape5 = (batch, _CONTRAST, _COIL, _H, _W)
    return (lam.reshape(shape5), inv.reshape(shape5))


# speedup vs baseline: 1.0684x; 1.0684x over previous
import jax
import jax.numpy as jnp
from jax.experimental import pallas as pl
from jax.experimental.pallas import tpu as pltpu

_H = 320
_W = 320
_CONTRAST = 4
_COIL = 12
_COIL_CHUNK = 6
_SLOPE = 5.0
_CENTER = 10
_R = 4.0


def _fused_body(w_ref, noise_ref, im_ref, lam_ref, inv_ref, mask_ref):
    j = pl.program_id(1)

    @pl.when(j == 0)
    def _compute_mask():
        w = w_ref[0]
        prob = jax.nn.sigmoid(w * _SLOPE)
        rows = jax.lax.broadcasted_iota(jnp.int32, (_H, _W), 0)
        cols = jax.lax.broadcasted_iota(jnp.int32, (_H, _W), 1)
        cy0, cy1 = _H // 2 - _CENTER // 2, _H // 2 + _CENTER // 2
        cx0, cx1 = _W // 2 - _CENTER // 2, _W // 2 + _CENTER // 2
        in_center = (rows >= cy0) & (rows < cy1) & (cols >= cx0) & (cols < cx1)
        p = jnp.where(in_center, 0.0, prob)
        s = jnp.sum(p)
        total = _H * _W / _R - _CENTER ** 2
        p_over = p * (total / s)
        inv_total = _H * _W * (1.0 - 1.0 / _R)
        inv_sum = _H * _W - s - _CENTER ** 2
        p_under = 1.0 - (1.0 - p) * (inv_total / inv_sum)
        p_new = jnp.where(s > total, p_over, p_under)
        p_new = jnp.where(in_center, 1.0, p_new)
        mask_ref[...] = (p_new - noise_ref[0] >= 0.0).astype(jnp.float32)

    m = mask_ref[...]
    im = im_ref[0]
    lam = im * m[None, :, :]
    lam_ref[0] = lam
    inv_ref[0] = im - lam


def kernel(undersampled_k, initial_mask, sampling_weights):
    batch = undersampled_k.shape[0]
    noise = jax.random.uniform(
        jax.random.key(42), (batch, _CONTRAST, _H, _W), dtype=jnp.float32
    )
    bc = batch * _CONTRAST
    im = initial_mask.reshape(bc, _COIL, _H, _W)
    noise_f = noise.reshape(bc, _H, _W)
    ncs = _COIL // _COIL_CHUNK

    lam, inv = pl.pallas_call(
        _fused_body,
        grid=(bc, ncs),
        in_specs=[
            pl.BlockSpec((1, _H, _W), lambda i, j: (i % _CONTRAST, 0, 0)),
            pl.BlockSpec((1, _H, _W), lambda i, j: (i, 0, 0)),
            pl.BlockSpec((1, _COIL_CHUNK, _H, _W), lambda i, j: (i, j, 0, 0)),
        ],
        out_specs=[
            pl.BlockSpec((1, _COIL_CHUNK, _H, _W), lambda i, j: (i, j, 0, 0)),
            pl.BlockSpec((1, _COIL_CHUNK, _H, _W), lambda i, j: (i, j, 0, 0)),
        ],
        out_shape=[
            jax.ShapeDtypeStruct((bc, _COIL, _H, _W), jnp.float32),
            jax.ShapeDtypeStruct((bc, _COIL, _H, _W), jnp.float32),
        ],
        scratch_shapes=[pltpu.VMEM((_H, _W), jnp.float32)],
        compiler_params=pltpu.CompilerParams(
            dimension_semantics=("parallel", "arbitrary"),
        ),
    )(sampling_weights, noise_f, im)

    shape5 = (batch, _CONTRAST, _COIL, _H, _W)
    return (lam.reshape(shape5), inv.reshape(shape5))


# speedup vs baseline: 1.0950x; 1.0249x over previous
import jax
import jax.numpy as jnp
from jax.experimental import pallas as pl
from jax.experimental.pallas import tpu as pltpu

_H = 320
_W = 320
_CONTRAST = 4
_COIL = 12
_COIL_CHUNK = 12
_SLOPE = 5.0
_CENTER = 10
_R = 4.0


def _fused_body(w_ref, noise_ref, im_ref, lam_ref, inv_ref, mask_ref):
    j = pl.program_id(1)

    @pl.when(j == 0)
    def _compute_mask():
        w = w_ref[0]
        prob = jax.nn.sigmoid(w * _SLOPE)
        rows = jax.lax.broadcasted_iota(jnp.int32, (_H, _W), 0)
        cols = jax.lax.broadcasted_iota(jnp.int32, (_H, _W), 1)
        cy0, cy1 = _H // 2 - _CENTER // 2, _H // 2 + _CENTER // 2
        cx0, cx1 = _W // 2 - _CENTER // 2, _W // 2 + _CENTER // 2
        in_center = (rows >= cy0) & (rows < cy1) & (cols >= cx0) & (cols < cx1)
        p = jnp.where(in_center, 0.0, prob)
        s = jnp.sum(p)
        total = _H * _W / _R - _CENTER ** 2
        p_over = p * (total / s)
        inv_total = _H * _W * (1.0 - 1.0 / _R)
        inv_sum = _H * _W - s - _CENTER ** 2
        p_under = 1.0 - (1.0 - p) * (inv_total / inv_sum)
        p_new = jnp.where(s > total, p_over, p_under)
        p_new = jnp.where(in_center, 1.0, p_new)
        mask_ref[...] = (p_new - noise_ref[0] >= 0.0).astype(jnp.float32)

    m = mask_ref[...]
    im = im_ref[0]
    lam = im * m[None, :, :]
    lam_ref[0] = lam
    inv_ref[0] = im - lam


def kernel(undersampled_k, initial_mask, sampling_weights):
    batch = undersampled_k.shape[0]
    noise = jax.random.uniform(
        jax.random.key(42), (batch, _CONTRAST, _H, _W), dtype=jnp.float32
    )
    bc = batch * _CONTRAST
    im = initial_mask.reshape(bc, _COIL, _H, _W)
    noise_f = noise.reshape(bc, _H, _W)
    ncs = _COIL // _COIL_CHUNK

    lam, inv = pl.pallas_call(
        _fused_body,
        grid=(bc, ncs),
        in_specs=[
            pl.BlockSpec((1, _H, _W), lambda i, j: (i % _CONTRAST, 0, 0)),
            pl.BlockSpec((1, _H, _W), lambda i, j: (i, 0, 0)),
            pl.BlockSpec((1, _COIL_CHUNK, _H, _W), lambda i, j: (i, j, 0, 0)),
        ],
        out_specs=[
            pl.BlockSpec((1, _COIL_CHUNK, _H, _W), lambda i, j: (i, j, 0, 0)),
            pl.BlockSpec((1, _COIL_CHUNK, _H, _W), lambda i, j: (i, j, 0, 0)),
        ],
        out_shape=[
            jax.ShapeDtypeStruct((bc, _COIL, _H, _W), jnp.float32),
            jax.ShapeDtypeStruct((bc, _COIL, _H, _W), jnp.float32),
        ],
        scratch_shapes=[pltpu.VMEM((_H, _W), jnp.float32)],
        compiler_params=pltpu.CompilerParams(
            dimension_semantics=("parallel", "arbitrary"),
        ),
    )(sampling_weights, noise_f, im)

    shape5 = (batch, _CONTRAST, _COIL, _H, _W)
    return (lam.reshape(shape5), inv.reshape(shape5))
